# baseline (speedup 1.0000x reference)
import functools

import jax
import jax.numpy as jnp
from jax import lax
from jax.experimental import pallas as pl
from jax.experimental.pallas import tpu as pltpu
from jax.experimental.pallas import tpu_sc as plsc

N_ROWS = 2_000_000
N_CLS = 10
BINS = 10
LANES = 16
NC, NS = 2, 16
NW = NC * NS
BLK = 4096
NBLK = -(-N_ROWS // BLK)
CHUNK = 4000
NCHUNKS = N_ROWS // CHUNK
GROUPS = CHUNK // LANES
CH_BASE = NCHUNKS // NW
CH_EXTRA = NCHUNKS - CH_BASE * NW


def _tc_math_body(x_ref, t_ref, w_ref, ce_ref, bin_ref):
    xb = x_ref[...]
    tb = t_ref[...]
    wv = w_ref[...]
    oh = jax.lax.broadcasted_iota(jnp.int32, (BLK, N_CLS), 1) == tb[:, None]
    m = jnp.max(xb, axis=-1)
    s = jnp.sum(jnp.exp(xb - m[:, None]), axis=-1)
    xt = jnp.sum(jnp.where(oh, xb, 0.0), axis=-1)
    wt = jnp.sum(jnp.where(oh, wv, 0.0), axis=-1)
    log_pt = xt - m - jnp.log(s)
    pt = jnp.exp(log_pt)
    ce = -(wt * log_pt)
    g = jnp.abs(pt - 1.0)
    bi = (g * jnp.float32(BINS - 0.0001)).astype(jnp.int32)
    bi = jnp.minimum(jnp.maximum(bi, 0), BINS - 1)
    ce_ref[...] = ce
    bin_ref[...] = bi.astype(jnp.float32)


_tc_math = pl.pallas_call(
    _tc_math_body,
    grid=(NBLK,),
    in_specs=[
        pl.BlockSpec((BLK, N_CLS), lambda i: (i, 0)),
        pl.BlockSpec((BLK,), lambda i: (i,)),
        pl.BlockSpec((1, N_CLS), lambda i: (0, 0)),
    ],
    out_specs=[
        pl.BlockSpec((BLK,), lambda i: (i,)),
        pl.BlockSpec((BLK,), lambda i: (i,)),
    ],
    out_shape=[
        jax.ShapeDtypeStruct((N_ROWS,), jnp.float32),
        jax.ShapeDtypeStruct((N_ROWS,), jnp.float32),
    ],
)


def _sc_body(ce_hbm, bin_hbm, out_hbm,
             cb0, cb1, bb0, bb1, cnt, sm, red,
             sc0, sc1, sb0, sb1):
    wid = lax.axis_index("s") * NC + lax.axis_index("c")
    nch = CH_BASE + jnp.where(wid < CH_EXTRA, 1, 0)

    zero16 = jnp.zeros((LANES,), jnp.float32)
    for i in range(LANES):
        cnt[pl.ds(i * LANES, LANES)] = zero16
        sm[pl.ds(i * LANES, LANES)] = zero16

    cbufs = (cb0, cb1)
    bbufs = (bb0, bb1)
    scs = (sc0, sc1)
    sbs = (sb0, sb1)

    lane = lax.iota(jnp.int32, LANES)
    ones = jnp.full((LANES,), 1.0, jnp.float32)

    def start(k, b):
        r0 = (wid + NW * k) * CHUNK
        pltpu.async_copy(ce_hbm.at[pl.ds(r0, CHUNK)], cbufs[b], scs[b])
        pltpu.async_copy(bin_hbm.at[pl.ds(r0, CHUNK)], bbufs[b], sbs[b])

    def wait(k, b):
        r0 = (wid + NW * k) * CHUNK
        pltpu.make_async_copy(
            ce_hbm.at[pl.ds(r0, CHUNK)], cbufs[b], scs[b]).wait()
        pltpu.make_async_copy(
            bin_hbm.at[pl.ds(r0, CHUNK)], bbufs[b], sbs[b]).wait()

    def process(b):
        cb, bb = cbufs[b], bbufs[b]

        def group(g):
            ce = cb[pl.ds(g * LANES, LANES)]
            bi = bb[pl.ds(g * LANES, LANES)].astype(jnp.int32)
            slot = lane * LANES + bi
            plsc.addupdate_scatter(cnt, [slot], ones)
            plsc.addupdate_scatter(sm, [slot], ce)

        plsc.parallel_loop(0, GROUPS, 1, unroll=8)(group)

    start(0, 0)

    def pair(i, carry):
        k0 = 2 * i

        @pl.when(k0 < nch)
        def _():
            wait(k0, 0)

            @pl.when(k0 + 1 < nch)
            def _():
                start(k0 + 1, 1)

            process(0)

        @pl.when(k0 + 1 < nch)
        def _():
            wait(k0 + 1, 1)

            @pl.when(k0 + 2 < nch)
            def _():
                start(k0 + 2, 0)

            process(1)

        return carry

    lax.fori_loop(0, (CH_BASE + 2) // 2, pair, 0)

    acc_c = cnt[pl.ds(0, LANES)]
    acc_s = sm[pl.ds(0, LANES)]
    for i in range(1, LANES):
        acc_c = acc_c + cnt[pl.ds(i * LANES, LANES)]
        acc_s = acc_s + sm[pl.ds(i * LANES, LANES)]
    red[pl.ds(0, LANES)] = acc_c
    red[pl.ds(LANES, LANES)] = acc_s
    pltpu.sync_copy(red.at[pl.ds(0, LANES)],
                    out_hbm.at[pl.ds(wid * LANES, LANES)])
    pltpu.sync_copy(red.at[pl.ds(LANES, LANES)],
                    out_hbm.at[pl.ds(NW * LANES + wid * LANES, LANES)])


_sc_hist = pl.kernel(
    _sc_body,
    out_type=jax.ShapeDtypeStruct((2 * NW * LANES,), jnp.float32),
    mesh=plsc.VectorSubcoreMesh(
        core_axis_name="c", subcore_axis_name="s",
        num_cores=NC, num_subcores=NS),
    compiler_params=pltpu.CompilerParams(needs_layout_passes=False),
    scratch_types=[
        pltpu.VMEM((CHUNK,), jnp.float32),
        pltpu.VMEM((CHUNK,), jnp.float32),
        pltpu.VMEM((CHUNK,), jnp.float32),
        pltpu.VMEM((CHUNK,), jnp.float32),
        pltpu.VMEM((LANES * LANES,), jnp.float32),
        pltpu.VMEM((LANES * LANES,), jnp.float32),
        pltpu.VMEM((2 * LANES,), jnp.float32),
        pltpu.SemaphoreType.DMA,
        pltpu.SemaphoreType.DMA,
        pltpu.SemaphoreType.DMA,
        pltpu.SemaphoreType.DMA,
    ],
)


def _fin_body(p_ref, o_ref):
    p = p_ref[...]
    c = jnp.sum(p[0], axis=0)
    s = jnp.sum(p[1], axis=0)
    k = jnp.sum(jnp.where(c > 0, 1.0, 0.0))
    gd = jnp.maximum(c * k, 1e-4)
    loss = jnp.sum(s / gd) * jnp.float32(1.0 / N_ROWS)
    o_ref[...] = jnp.reshape(loss, (1, 1))


@functools.partial(jax.jit)
def kernel(x, target, weight, stage):
    w_eff = jnp.where(stage == 1, jnp.ones_like(weight), weight)
    ce, bins = _tc_math(x, target, w_eff.reshape(1, N_CLS))
    partials = _sc_hist(ce, bins)
    loss = pl.pallas_call(
        _fin_body,
        out_shape=jax.ShapeDtypeStruct((1, 1), jnp.float32),
    )(partials.reshape(2, NW, LANES))
    return loss[0, 0]


# speedup vs baseline: 4.0052x; 4.0052x over previous
import functools

import jax
import jax.numpy as jnp
from jax import lax
from jax.experimental import pallas as pl
from jax.experimental.pallas import tpu as pltpu
from jax.experimental.pallas import tpu_sc as plsc

N_ROWS = 2_000_000
N_CLS = 10
BINS = 10
LANES = 16
NC, NS = 2, 16
NW = NC * NS
NSLICES = 4
SLICE = N_ROWS // NSLICES
CHUNK = 384
GROUPS = CHUNK // LANES
NCHUNKS = -(-SLICE // CHUNK)
CH_BASE = NCHUNKS // NW
CH_EXTRA = NCHUNKS - CH_BASE * NW

_LN2 = 0.6931472
_SQRT2 = 1.4142135


def _log_f32(s):
    bits = plsc.bitcast(s, jnp.int32)
    e = (bits >> 23) - 127
    mant = plsc.bitcast((bits & 0x007FFFFF) | 0x3F800000, jnp.float32)
    big = mant > _SQRT2
    mant = jnp.where(big, mant * 0.5, mant)
    ef = (e + jnp.where(big, 1, 0)).astype(jnp.float32)
    z = (mant - 1.0) / (mant + 1.0)
    zz = z * z
    p = jnp.float32(1.0 / 7.0)
    p = p * zz + jnp.float32(1.0 / 5.0)
    p = p * zz + jnp.float32(1.0 / 3.0)
    p = p * zz + 1.0
    return ef * _LN2 + 2.0 * z * p


def _sc_body(x_hbm, t_hbm, w_hbm, out_hbm,
             xb0, xb1, tb0, tb1, wv, cnt, sm, red,
             sx0, sx1, st0, st1):
    wid = lax.axis_index("s") * NC + lax.axis_index("c")
    nch = CH_BASE + jnp.where(wid < CH_EXTRA, 1, 0)

    pltpu.sync_copy(w_hbm, wv)
    zero16 = jnp.zeros((LANES,), jnp.float32)
    for i in range(LANES):
        cnt[pl.ds(i * LANES, LANES)] = zero16
        sm[pl.ds(i * LANES, LANES)] = zero16

    xbufs = (xb0, xb1)
    tbufs = (tb0, tb1)
    sxs = (sx0, sx1)
    sts = (st0, st1)

    lane = lax.iota(jnp.int32, LANES)
    ones = jnp.full((LANES,), 1.0, jnp.float32)

    def start(k, b):
        cid = wid + NW * k
        r0 = cid * CHUNK
        off = jnp.minimum(r0, SLICE - CHUNK)
        pltpu.async_copy(t_hbm.at[pl.ds(off, CHUNK)], tbufs[b], sts[b])
        pltpu.async_copy(x_hbm.at[pl.ds(off, CHUNK), :], xbufs[b], sxs[b])

    def wait(k, b):
        cid = wid + NW * k
        r0 = cid * CHUNK
        off = jnp.minimum(r0, SLICE - CHUNK)
        pltpu.make_async_copy(
            t_hbm.at[pl.ds(off, CHUNK)], tbufs[b], sts[b]).wait()
        pltpu.make_async_copy(
            x_hbm.at[pl.ds(off, CHUNK), :], xbufs[b], sxs[b]).wait()

    def _tree_reduce(vals, op):
        while len(vals) > 1:
            nxt = [op(vals[i], vals[i + 1]) for i in range(0, len(vals) - 1, 2)]
            if len(vals) % 2:
                nxt.append(vals[-1])
            vals = nxt
        return vals[0]

    def process(b, k):
        xb, tb = xbufs[b], tbufs[b]
        cid = wid + NW * k
        r0 = cid * CHUNK
        shift = r0 - jnp.minimum(r0, SLICE - CHUNK)

        def group(g):
            rglob = r0 + g * LANES + lane
            valid = rglob < SLICE
            tvec = tb[pl.ds(shift + g * LANES, LANES)]
            rloc = shift + g * LANES + lane
            cols = [plsc.load_gather(
                xb, [rloc, jnp.full((LANES,), c, jnp.int32)])
                for c in range(N_CLS)]
            m = _tree_reduce(cols, jnp.maximum)
            s = _tree_reduce([jnp.exp(c - m) for c in cols], jnp.add)
            xt = plsc.load_gather(xb, [rloc, tvec])
            wt = plsc.load_gather(wv, [tvec])
            et = jnp.exp(xt - m)
            pt = et / s
            log_pt = xt - m - _log_f32(s)
            ce = -(wt * log_pt)
            g_norm = jnp.abs(pt - 1.0)
            bi = (g_norm * jnp.float32(BINS - 0.0001)).astype(jnp.int32)
            bi = jnp.minimum(jnp.maximum(bi, 0), BINS - 1)
            slot = lane * LANES + bi
            plsc.addupdate_scatter(cnt, [slot], ones, mask=valid)
            plsc.addupdate_scatter(sm, [slot], ce, mask=valid)

        plsc.parallel_loop(0, GROUPS, 1, unroll=4)(group)

    start(0, 0)

    def pair(i, carry):
        k0 = 2 * i

        @pl.when(k0 < nch)
        def _():
            wait(k0, 0)

            @pl.when(k0 + 1 < nch)
            def _():
                start(k0 + 1, 1)

            process(0, k0)

        @pl.when(k0 + 1 < nch)
        def _():
            wait(k0 + 1, 1)

            @pl.when(k0 + 2 < nch)
            def _():
                start(k0 + 2, 0)

            process(1, k0 + 1)

        return carry

    lax.fori_loop(0, (CH_BASE + 2) // 2, pair, 0)

    acc_c = cnt[pl.ds(0, LANES)]
    acc_s = sm[pl.ds(0, LANES)]
    for i in range(1, LANES):
        acc_c = acc_c + cnt[pl.ds(i * LANES, LANES)]
        acc_s = acc_s + sm[pl.ds(i * LANES, LANES)]
    red[pl.ds(0, LANES)] = acc_c
    red[pl.ds(LANES, LANES)] = acc_s
    pltpu.sync_copy(red.at[pl.ds(0, LANES)],
                    out_hbm.at[pl.ds(wid * LANES, LANES)])
    pltpu.sync_copy(red.at[pl.ds(LANES, LANES)],
                    out_hbm.at[pl.ds(NW * LANES + wid * LANES, LANES)])


_sc_pass = pl.kernel(
    _sc_body,
    out_type=jax.ShapeDtypeStruct((2 * NW * LANES,), jnp.float32),
    mesh=plsc.VectorSubcoreMesh(
        core_axis_name="c", subcore_axis_name="s",
        num_cores=NC, num_subcores=NS),
    compiler_params=pltpu.CompilerParams(needs_layout_passes=False),
    scratch_types=[
        pltpu.VMEM((CHUNK, N_CLS), jnp.float32),
        pltpu.VMEM((CHUNK, N_CLS), jnp.float32),
        pltpu.VMEM((CHUNK,), jnp.int32),
        pltpu.VMEM((CHUNK,), jnp.int32),
        pltpu.VMEM((LANES,), jnp.float32),
        pltpu.VMEM((LANES * LANES,), jnp.float32),
        pltpu.VMEM((LANES * LANES,), jnp.float32),
        pltpu.VMEM((2 * LANES,), jnp.float32),
        pltpu.SemaphoreType.DMA,
        pltpu.SemaphoreType.DMA,
        pltpu.SemaphoreType.DMA,
        pltpu.SemaphoreType.DMA,
    ],
)


def _fin_body(p_ref, o_ref):
    p = p_ref[...]
    c = jnp.sum(sum(p[2 * i] for i in range(NSLICES)), axis=0)
    s = jnp.sum(sum(p[2 * i + 1] for i in range(NSLICES)), axis=0)
    k = jnp.sum(jnp.where(c > 0, 1.0, 0.0))
    gd = jnp.maximum(c * k, 1e-4)
    loss = jnp.sum(s / gd) * jnp.float32(1.0 / N_ROWS)
    o_ref[...] = jnp.reshape(loss, (1, 1))


@functools.partial(jax.jit)
def kernel(x, target, weight, stage):
    w_eff = jnp.where(stage == 1, jnp.ones_like(weight), weight)
    w16 = jnp.zeros((LANES,), jnp.float32).at[:N_CLS].set(w_eff)
    parts = []
    for i in range(NSLICES):
        p = _sc_pass(
            lax.slice(x, (i * SLICE, 0), ((i + 1) * SLICE, N_CLS)),
            lax.slice(target, (i * SLICE,), ((i + 1) * SLICE,)),
            w16)
        parts.append(p.reshape(2, NW, LANES))
    loss = pl.pallas_call(
        _fin_body,
        out_shape=jax.ShapeDtypeStruct((1, 1), jnp.float32),
    )(jnp.concatenate(parts, axis=0))
    return loss[0, 0]
